# baseline (device time: 183193 ns/iter reference)
import jax
import jax.numpy as jnp
from jax import lax
from jax.experimental import pallas as pl
from jax.experimental.pallas import tpu as pltpu

N_DEV = 4
SQ = 1024
SKV = 1024
HQ_LOCAL = 8
DH = 128
D_MODEL = 1024
BLK = 64
SCALE = 0.08838834764831843
NEG = -1e9


def kernel(x, Wq, K_ext, V_ext, Wo):
    def body(x_ref, wq_ref, k_ref, v_ref, wo_ref, out_ref,
             kv_buf, kv_send_buf, stage, q_buf, comm, ar_bf, ag_comm, ag0_bf,
             kv_send_sems, kv_recv_sems, stage_sems,
             ar_send_sems, ar_recv_sems, ag_send_sems, ag_recv_sems):
        my = lax.axis_index("i")
        right = (my + 1) % N_DEV

        bsem = pltpu.get_barrier_semaphore()
        for d in (1, 2, 3):
            pl.semaphore_signal(
                bsem, inc=1,
                device_id=((my + d) % N_DEV,),
                device_id_type=pl.DeviceIdType.MESH,
            )
        pl.semaphore_wait(bsem, N_DEV - 1)

        def kv_rdma(j, t):
            return pltpu.make_async_remote_copy(
                src_ref=kv_send_buf.at[t, :, (j - 1) * HQ_LOCAL:j * HQ_LOCAL, :],
                dst_ref=kv_buf.at[t],
                send_sem=kv_send_sems.at[j - 1, t],
                recv_sem=kv_recv_sems.at[t],
                device_id=(j,),
                device_id_type=pl.DeviceIdType.MESH,
            )

        @pl.when(my == 0)
        def _():
            jobs = [(j, t) for j in (2, 1, 3, 0) for t in (0, 1)]

            def stage_dma(idx, slot):
                j, t = jobs[idx]
                src = k_ref if t == 0 else v_ref
                return pltpu.make_async_copy(
                    src.at[0, :, j * HQ_LOCAL:(j + 1) * HQ_LOCAL, :],
                    stage.at[slot],
                    stage_sems.at[slot],
                )

            stage_dma(0, 0).start()
            stage_dma(1, 1).start()
            for idx, (j, t) in enumerate(jobs):
                slot = idx % 2
                stage_dma(idx, slot).wait()
                bf = stage[slot].astype(jnp.bfloat16)
                if j == 0:
                    kv_buf[t] = bf
                else:
                    kv_send_buf[t, :, (j - 1) * HQ_LOCAL:j * HQ_LOCAL, :] = bf
                    kv_rdma(j, t).start()
                if idx + 2 < len(jobs):
                    stage_dma(idx + 2, slot).start()

        q_buf[...] = jnp.dot(x_ref[0], wq_ref[...],
                             preferred_element_type=jnp.float32)

        @pl.when(my != 0)
        def _():
            for t in (0, 1):
                kv_rdma(1, t).wait_recv()

        @pl.when(my == 0)
        def _():
            for j in (1, 2, 3):
                for t in (0, 1):
                    kv_rdma(j, t).wait_send()

        rows = SQ // N_DEV

        def rs_rdma(hop):
            return pltpu.make_async_remote_copy(
                src_ref=ar_bf.at[hop],
                dst_ref=comm.at[hop],
                send_sem=ar_send_sems.at[hop],
                recv_sem=ar_recv_sems.at[hop],
                device_id=(right,),
                device_id_type=pl.DeviceIdType.MESH,
            )

        def ag_rdma(hop):
            return pltpu.make_async_remote_copy(
                src_ref=ag0_bf.at[0] if hop == 0 else ag_comm.at[hop - 1],
                dst_ref=ag_comm.at[hop],
                send_sem=ag_send_sems.at[hop],
                recv_sem=ag_recv_sems.at[hop],
                device_id=(right,),
                device_id_type=pl.DeviceIdType.MESH,
            )

        for c in range(N_DEV):
            g = (my - c) % N_DEV
            off = g * rows
            qc = q_buf[pl.ds(off, rows), :]
            qb_c = (
                lax.broadcasted_iota(jnp.int32, (rows, SKV), 0) + off
            ) // BLK
            kb_c = lax.broadcasted_iota(jnp.int32, (rows, SKV), 1) // BLK
            mask_c = kb_c <= qb_c
            part = jnp.zeros((rows, D_MODEL), jnp.float32)
            for h in range(HQ_LOCAL):
                qh = qc[:, h * DH:(h + 1) * DH]
                kh = kv_buf[0, :, h, :].astype(jnp.float32)
                vh = kv_buf[1, :, h, :].astype(jnp.float32)
                s = lax.dot_general(
                    qh, kh, (((1,), (1,)), ((), ())),
                    preferred_element_type=jnp.float32,
                ) * SCALE
                s = jnp.where(mask_c, s, NEG)
                m = jnp.max(s, axis=1, keepdims=True)
                e = jnp.exp(s - m)
                w = e / jnp.sum(e, axis=1, keepdims=True)
                ctx_h = jnp.dot(w, vh, preferred_element_type=jnp.float32)
                part = part + jnp.dot(
                    ctx_h, wo_ref[h * DH:(h + 1) * DH, :],
                    preferred_element_type=jnp.float32,
                )
            if c == 0:
                ar_bf[0] = part.astype(jnp.bfloat16)
                rs_rdma(0).start()
            else:
                rs_rdma(c - 1).wait_recv()
                part = part + comm[c - 1].astype(jnp.float32)
                if c < N_DEV - 1:
                    ar_bf[c] = part.astype(jnp.bfloat16)
                    rs_rdma(c).start()
                else:
                    out_ref[0, pl.ds(off, rows), :] = part
                    ag0_bf[0] = part.astype(jnp.bfloat16)
        for hop in range(N_DEV - 1):
            g = (my - hop) % N_DEV
            rdma = ag_rdma(hop)
            rdma.start()
            rdma.wait_recv()
            out_ref[0, pl.ds(g * rows, rows), :] = (
                ag_comm[hop].astype(jnp.float32)
            )
        for hop in range(N_DEV - 1):
            rs_rdma(hop).wait_send()
            ag_rdma(hop).wait_send()

    return pl.pallas_call(
        body,
        out_shape=jax.ShapeDtypeStruct((1, SQ, D_MODEL), jnp.float32),
        in_specs=[
            pl.BlockSpec(memory_space=pltpu.VMEM),
            pl.BlockSpec(memory_space=pltpu.VMEM),
            pl.BlockSpec(memory_space=pl.ANY),
            pl.BlockSpec(memory_space=pl.ANY),
            pl.BlockSpec(memory_space=pltpu.VMEM),
        ],
        out_specs=pl.BlockSpec(memory_space=pltpu.VMEM),
        scratch_shapes=[
            pltpu.VMEM((2, SKV, HQ_LOCAL, DH), jnp.bfloat16),
            pltpu.VMEM((2, SKV, 3 * HQ_LOCAL, DH), jnp.bfloat16),
            pltpu.VMEM((2, SKV, HQ_LOCAL, DH), jnp.float32),
            pltpu.VMEM((SQ, D_MODEL), jnp.float32),
            pltpu.VMEM((N_DEV - 1, SQ // N_DEV, D_MODEL), jnp.bfloat16),
            pltpu.VMEM((N_DEV - 1, SQ // N_DEV, D_MODEL), jnp.bfloat16),
            pltpu.VMEM((N_DEV - 1, SQ // N_DEV, D_MODEL), jnp.bfloat16),
            pltpu.VMEM((1, SQ // N_DEV, D_MODEL), jnp.bfloat16),
            pltpu.SemaphoreType.DMA((N_DEV - 1, 2)),
            pltpu.SemaphoreType.DMA((2,)),
            pltpu.SemaphoreType.DMA((2,)),
            pltpu.SemaphoreType.DMA((N_DEV - 1,)),
            pltpu.SemaphoreType.DMA((N_DEV - 1,)),
            pltpu.SemaphoreType.DMA((N_DEV - 1,)),
            pltpu.SemaphoreType.DMA((N_DEV - 1,)),
        ],
        compiler_params=pltpu.CompilerParams(
            collective_id=0,
            vmem_limit_bytes=63 * 1024 * 1024,
        ),
    )(x, Wq, K_ext, V_ext, Wo)


# device time: 162733 ns/iter; 1.1257x vs baseline; 1.1257x over previous
import jax
import jax.numpy as jnp
from jax import lax
from jax.experimental import pallas as pl
from jax.experimental.pallas import tpu as pltpu

N_DEV = 4
SQ = 1024
SKV = 1024
HQ_LOCAL = 8
DH = 128
D_MODEL = 1024
BLK = 64
SCALE = 0.08838834764831843
NEG = -1e9


def kernel(x, Wq, K_ext, V_ext, Wo):
    def body(x_ref, wq_ref, k_ref, v_ref, wo_ref, out_ref,
             kv_buf, kv_send_buf, stage, q_buf, relay_buf,
             comm, ar_bf, ag_comm, ag0_bf,
             kv_send_sems, kv_recv_sems, stage_sems,
             relay_recv_sems, relay_fwd_sems,
             ar_send_sems, ar_recv_sems, ag_send_sems, ag_recv_sems):
        my = lax.axis_index("i")
        right = (my + 1) % N_DEV

        bsem = pltpu.get_barrier_semaphore()
        for d in (1, 2, 3):
            pl.semaphore_signal(
                bsem, inc=1,
                device_id=((my + d) % N_DEV,),
                device_id_type=pl.DeviceIdType.MESH,
            )
        pl.semaphore_wait(bsem, N_DEV - 1)

        def kv_rdma(j, t):
            if j == 2:
                dst = relay_buf
                dev = 1 if t == 0 else 3
                recv = relay_recv_sems.at[0]
            else:
                dst = kv_buf.at[t]
                dev = j
                recv = kv_recv_sems.at[t]
            return pltpu.make_async_remote_copy(
                src_ref=kv_send_buf.at[t, :, (j - 1) * HQ_LOCAL:j * HQ_LOCAL, :],
                dst_ref=dst,
                send_sem=kv_send_sems.at[j - 1, t],
                recv_sem=recv,
                device_id=(dev,),
                device_id_type=pl.DeviceIdType.MESH,
            )

        def fwd_rdma(t):
            return pltpu.make_async_remote_copy(
                src_ref=relay_buf,
                dst_ref=kv_buf.at[t],
                send_sem=relay_fwd_sems.at[0],
                recv_sem=kv_recv_sems.at[t],
                device_id=(2,),
                device_id_type=pl.DeviceIdType.MESH,
            )

        @pl.when(my == 0)
        def _():
            jobs = [(2, 0), (2, 1), (1, 0), (3, 0), (1, 1), (3, 1),
                    (0, 0), (0, 1)]

            def stage_dma(idx, slot):
                j, t = jobs[idx]
                src = k_ref if t == 0 else v_ref
                return pltpu.make_async_copy(
                    src.at[0, :, j * HQ_LOCAL:(j + 1) * HQ_LOCAL, :],
                    stage.at[slot],
                    stage_sems.at[slot],
                )

            stage_dma(0, 0).start()
            stage_dma(1, 1).start()
            for idx, (j, t) in enumerate(jobs):
                slot = idx % 2
                stage_dma(idx, slot).wait()
                bf = stage[slot].astype(jnp.bfloat16)
                if j == 0:
                    kv_buf[t] = bf
                else:
                    kv_send_buf[t, :, (j - 1) * HQ_LOCAL:j * HQ_LOCAL, :] = bf
                    kv_rdma(j, t).start()
                if idx + 2 < len(jobs):
                    stage_dma(idx + 2, slot).start()

        @pl.when(my == 1)
        def _():
            kv_rdma(2, 0).wait_recv()
            fwd_rdma(0).start()

        @pl.when(my == 3)
        def _():
            kv_rdma(2, 1).wait_recv()
            fwd_rdma(1).start()

        q_buf[...] = jnp.dot(x_ref[0], wq_ref[...],
                             preferred_element_type=jnp.float32)

        @pl.when(my != 0)
        def _():
            for t in (0, 1):
                kv_rdma(1, t).wait_recv()

        @pl.when(my == 0)
        def _():
            for j in (1, 2, 3):
                for t in (0, 1):
                    kv_rdma(j, t).wait_send()

        @pl.when(my == 1)
        def _():
            fwd_rdma(0).wait_send()

        @pl.when(my == 3)
        def _():
            fwd_rdma(1).wait_send()

        rows = SQ // N_DEV

        def rs_rdma(hop):
            return pltpu.make_async_remote_copy(
                src_ref=ar_bf.at[hop],
                dst_ref=comm.at[hop],
                send_sem=ar_send_sems.at[hop],
                recv_sem=ar_recv_sems.at[hop],
                device_id=(right,),
                device_id_type=pl.DeviceIdType.MESH,
            )

        def ag_rdma(hop):
            return pltpu.make_async_remote_copy(
                src_ref=ag0_bf.at[0] if hop == 0 else ag_comm.at[hop - 1],
                dst_ref=ag_comm.at[hop],
                send_sem=ag_send_sems.at[hop],
                recv_sem=ag_recv_sems.at[hop],
                device_id=(right,),
                device_id_type=pl.DeviceIdType.MESH,
            )

        for c in range(N_DEV):
            g = (my - c) % N_DEV
            off = g * rows
            qc = q_buf[pl.ds(off, rows), :]
            qb_c = (
                lax.broadcasted_iota(jnp.int32, (rows, SKV), 0) + off
            ) // BLK
            kb_c = lax.broadcasted_iota(jnp.int32, (rows, SKV), 1) // BLK
            mask_c = kb_c <= qb_c
            part = jnp.zeros((rows, D_MODEL), jnp.float32)
            for h in range(HQ_LOCAL):
                qh = qc[:, h * DH:(h + 1) * DH]
                kh = kv_buf[0, :, h, :].astype(jnp.float32)
                vh = kv_buf[1, :, h, :].astype(jnp.float32)
                s = lax.dot_general(
                    qh, kh, (((1,), (1,)), ((), ())),
                    preferred_element_type=jnp.float32,
                ) * SCALE
                s = jnp.where(mask_c, s, NEG)
                m = jnp.max(s, axis=1, keepdims=True)
                e = jnp.exp(s - m)
                w = e / jnp.sum(e, axis=1, keepdims=True)
                ctx_h = jnp.dot(w, vh, preferred_element_type=jnp.float32)
                part = part + jnp.dot(
                    ctx_h, wo_ref[h * DH:(h + 1) * DH, :],
                    preferred_element_type=jnp.float32,
                )
            if c == 0:
                ar_bf[0] = part.astype(jnp.bfloat16)
                rs_rdma(0).start()
            else:
                rs_rdma(c - 1).wait_recv()
                part = part + comm[c - 1].astype(jnp.float32)
                if c < N_DEV - 1:
                    ar_bf[c] = part.astype(jnp.bfloat16)
                    rs_rdma(c).start()
                else:
                    out_ref[0, pl.ds(off, rows), :] = part
                    ag0_bf[0] = part.astype(jnp.bfloat16)
        for hop in range(N_DEV - 1):
            g = (my - hop) % N_DEV
            rdma = ag_rdma(hop)
            rdma.start()
            rdma.wait_recv()
            out_ref[0, pl.ds(g * rows, rows), :] = (
                ag_comm[hop].astype(jnp.float32)
            )
        for hop in range(N_DEV - 1):
            rs_rdma(hop).wait_send()
            ag_rdma(hop).wait_send()

    return pl.pallas_call(
        body,
        out_shape=jax.ShapeDtypeStruct((1, SQ, D_MODEL), jnp.float32),
        in_specs=[
            pl.BlockSpec(memory_space=pltpu.VMEM),
            pl.BlockSpec(memory_space=pltpu.VMEM),
            pl.BlockSpec(memory_space=pl.ANY),
            pl.BlockSpec(memory_space=pl.ANY),
            pl.BlockSpec(memory_space=pltpu.VMEM),
        ],
        out_specs=pl.BlockSpec(memory_space=pltpu.VMEM),
        scratch_shapes=[
            pltpu.VMEM((2, SKV, HQ_LOCAL, DH), jnp.bfloat16),
            pltpu.VMEM((2, SKV, 3 * HQ_LOCAL, DH), jnp.bfloat16),
            pltpu.VMEM((2, SKV, HQ_LOCAL, DH), jnp.float32),
            pltpu.VMEM((SQ, D_MODEL), jnp.float32),
            pltpu.VMEM((SKV, HQ_LOCAL, DH), jnp.bfloat16),
            pltpu.VMEM((N_DEV - 1, SQ // N_DEV, D_MODEL), jnp.bfloat16),
            pltpu.VMEM((N_DEV - 1, SQ // N_DEV, D_MODEL), jnp.bfloat16),
            pltpu.VMEM((N_DEV - 1, SQ // N_DEV, D_MODEL), jnp.bfloat16),
            pltpu.VMEM((1, SQ // N_DEV, D_MODEL), jnp.bfloat16),
            pltpu.SemaphoreType.DMA((N_DEV - 1, 2)),
            pltpu.SemaphoreType.DMA((2,)),
            pltpu.SemaphoreType.DMA((2,)),
            pltpu.SemaphoreType.DMA((1,)),
            pltpu.SemaphoreType.DMA((1,)),
            pltpu.SemaphoreType.DMA((N_DEV - 1,)),
            pltpu.SemaphoreType.DMA((N_DEV - 1,)),
            pltpu.SemaphoreType.DMA((N_DEV - 1,)),
            pltpu.SemaphoreType.DMA((N_DEV - 1,)),
        ],
        compiler_params=pltpu.CompilerParams(
            collective_id=0,
            vmem_limit_bytes=63 * 1024 * 1024,
        ),
    )(x, Wq, K_ext, V_ext, Wo)


# device time: 150095 ns/iter; 1.2205x vs baseline; 1.0842x over previous
import jax
import jax.numpy as jnp
from jax import lax
from jax.experimental import pallas as pl
from jax.experimental.pallas import tpu as pltpu

N_DEV = 4
SQ = 1024
SKV = 1024
HQ_LOCAL = 8
DH = 128
D_MODEL = 1024
BLK = 64
SCALE = 0.08838834764831843
NEG = -1e9


def kernel(x, Wq, K_ext, V_ext, Wo):
    def body(x_ref, wq_ref, k_ref, v_ref, wo_ref, out_ref,
             kv_buf, kv_send_buf, stage, q_buf,
             comm, ar_bf, ag_comm, ag0_bf,
             kv_send_sems, kv_recv_sems, stage_sems,
             relay_recv_sems, relay_fwd_sems,
             ar_send_sems, ar_recv_sems, ag_send_sems, ag_recv_sems):
        my = lax.axis_index("i")
        right = (my + 1) % N_DEV

        bsem = pltpu.get_barrier_semaphore()
        for d in (1, 2, 3):
            pl.semaphore_signal(
                bsem, inc=1,
                device_id=((my + d) % N_DEV,),
                device_id_type=pl.DeviceIdType.MESH,
            )
        pl.semaphore_wait(bsem, N_DEV - 1)

        def relay_view(t):
            return kv_send_buf.at[t, :, 0:HQ_LOCAL, :]

        def kv_rdma(j, t):
            if j == 2:
                dst = relay_view(t)
                dev = 1 if t == 0 else 3
                recv = relay_recv_sems.at[0]
            else:
                dst = kv_buf.at[t]
                dev = j
                recv = kv_recv_sems.at[t]
            return pltpu.make_async_remote_copy(
                src_ref=kv_send_buf.at[t, :, (j - 1) * HQ_LOCAL:j * HQ_LOCAL, :],
                dst_ref=dst,
                send_sem=kv_send_sems.at[j - 1, t],
                recv_sem=recv,
                device_id=(dev,),
                device_id_type=pl.DeviceIdType.MESH,
            )

        def fwd_rdma(t):
            return pltpu.make_async_remote_copy(
                src_ref=relay_view(t),
                dst_ref=kv_buf.at[t],
                send_sem=relay_fwd_sems.at[0],
                recv_sem=kv_recv_sems.at[t],
                device_id=(2,),
                device_id_type=pl.DeviceIdType.MESH,
            )

        @pl.when(my == 0)
        def _():
            jobs = [(2, 0), (2, 1), (1, 0), (3, 0), (1, 1), (3, 1),
                    (0, 0), (0, 1)]

            def stage_dma(idx, slot):
                j, t = jobs[idx]
                src = k_ref if t == 0 else v_ref
                return pltpu.make_async_copy(
                    src.at[0, :, j * HQ_LOCAL:(j + 1) * HQ_LOCAL, :],
                    stage.at[slot],
                    stage_sems.at[slot],
                )

            stage_dma(0, 0).start()
            stage_dma(1, 1).start()
            for idx, (j, t) in enumerate(jobs):
                slot = idx % 2
                stage_dma(idx, slot).wait()
                bf = stage[slot].astype(jnp.bfloat16)
                if j == 0:
                    kv_buf[t] = bf
                else:
                    kv_send_buf[t, :, (j - 1) * HQ_LOCAL:j * HQ_LOCAL, :] = bf
                    kv_rdma(j, t).start()
                if idx + 2 < len(jobs):
                    stage_dma(idx + 2, slot).start()

        @pl.when(my == 1)
        def _():
            kv_rdma(2, 0).wait_recv()
            fwd_rdma(0).start()

        @pl.when(my == 3)
        def _():
            kv_rdma(2, 1).wait_recv()
            fwd_rdma(1).start()

        q_buf[...] = jnp.dot(x_ref[0], wq_ref[...],
                             preferred_element_type=jnp.float32) * SCALE

        @pl.when(my != 0)
        def _():
            for t in (0, 1):
                kv_rdma(1, t).wait_recv()

        @pl.when(my == 0)
        def _():
            for j in (1, 2, 3):
                for t in (0, 1):
                    kv_rdma(j, t).wait_send()

        @pl.when(my == 1)
        def _():
            fwd_rdma(0).wait_send()

        @pl.when(my == 3)
        def _():
            fwd_rdma(1).wait_send()

        rows = SQ // N_DEV

        def rs_rdma(hop):
            return pltpu.make_async_remote_copy(
                src_ref=ar_bf.at[hop],
                dst_ref=comm.at[hop],
                send_sem=ar_send_sems.at[hop],
                recv_sem=ar_recv_sems.at[hop],
                device_id=(right,),
                device_id_type=pl.DeviceIdType.MESH,
            )

        def ag_rdma(hop):
            return pltpu.make_async_remote_copy(
                src_ref=ag0_bf.at[0] if hop == 0 else ag_comm.at[hop - 1],
                dst_ref=ag_comm.at[hop],
                send_sem=ag_send_sems.at[hop],
                recv_sem=ag_recv_sems.at[hop],
                device_id=(right,),
                device_id_type=pl.DeviceIdType.MESH,
            )

        qb = lax.broadcasted_iota(jnp.int32, (SQ, SKV), 0) // BLK
        kb = lax.broadcasted_iota(jnp.int32, (SQ, SKV), 1) // BLK
        mask = kb <= qb
        out_ref[0] = jnp.zeros((SQ, D_MODEL), jnp.float32)
        for h in range(HQ_LOCAL):
            qh = q_buf[:, h * DH:(h + 1) * DH]
            kh = kv_buf[0, :, h, :].astype(jnp.float32)
            vh = kv_buf[1, :, h, :].astype(jnp.float32)
            s = lax.dot_general(
                qh, kh, (((1,), (1,)), ((), ())),
                preferred_element_type=jnp.float32,
            )
            e = jnp.exp(jnp.where(mask, s, NEG))
            denom = jnp.sum(e, axis=1, keepdims=True)
            ctx_h = jnp.dot(
                e, vh, preferred_element_type=jnp.float32,
            ) * (1.0 / denom)
            out_ref[0] = out_ref[0] + jnp.dot(
                ctx_h, wo_ref[h * DH:(h + 1) * DH, :],
                preferred_element_type=jnp.float32,
            )

        for hop in range(N_DEV - 1):
            s_idx = ((my - hop) % N_DEV) * rows
            r_idx = ((my - hop - 1) % N_DEV) * rows
            ar_bf[hop] = out_ref[0, pl.ds(s_idx, rows), :].astype(jnp.bfloat16)
            rdma = rs_rdma(hop)
            rdma.start()
            rdma.wait_recv()
            out_ref[0, pl.ds(r_idx, rows), :] = (
                out_ref[0, pl.ds(r_idx, rows), :]
                + comm[hop].astype(jnp.float32)
            )
        ag0_bf[0] = out_ref[
            0, pl.ds(((my + 1) % N_DEV) * rows, rows), :
        ].astype(jnp.bfloat16)
        for hop in range(N_DEV - 1):
            g = (my - hop) % N_DEV
            rdma = ag_rdma(hop)
            rdma.start()
            rdma.wait_recv()
            out_ref[0, pl.ds(g * rows, rows), :] = (
                ag_comm[hop].astype(jnp.float32)
            )
        for hop in range(N_DEV - 1):
            rs_rdma(hop).wait_send()
            ag_rdma(hop).wait_send()

    return pl.pallas_call(
        body,
        out_shape=jax.ShapeDtypeStruct((1, SQ, D_MODEL), jnp.float32),
        in_specs=[
            pl.BlockSpec(memory_space=pltpu.VMEM),
            pl.BlockSpec(memory_space=pltpu.VMEM),
            pl.BlockSpec(memory_space=pl.ANY),
            pl.BlockSpec(memory_space=pl.ANY),
            pl.BlockSpec(memory_space=pltpu.VMEM),
        ],
        out_specs=pl.BlockSpec(memory_space=pltpu.VMEM),
        scratch_shapes=[
            pltpu.VMEM((2, SKV, HQ_LOCAL, DH), jnp.bfloat16),
            pltpu.VMEM((2, SKV, 3 * HQ_LOCAL, DH), jnp.bfloat16),
            pltpu.VMEM((2, SKV, HQ_LOCAL, DH), jnp.float32),
            pltpu.VMEM((SQ, D_MODEL), jnp.float32),
            pltpu.VMEM((N_DEV - 1, SQ // N_DEV, D_MODEL), jnp.bfloat16),
            pltpu.VMEM((N_DEV - 1, SQ // N_DEV, D_MODEL), jnp.bfloat16),
            pltpu.VMEM((N_DEV - 1, SQ // N_DEV, D_MODEL), jnp.bfloat16),
            pltpu.VMEM((1, SQ // N_DEV, D_MODEL), jnp.bfloat16),
            pltpu.SemaphoreType.DMA((N_DEV - 1, 2)),
            pltpu.SemaphoreType.DMA((2,)),
            pltpu.SemaphoreType.DMA((2,)),
            pltpu.SemaphoreType.DMA((1,)),
            pltpu.SemaphoreType.DMA((1,)),
            pltpu.SemaphoreType.DMA((N_DEV - 1,)),
            pltpu.SemaphoreType.DMA((N_DEV - 1,)),
            pltpu.SemaphoreType.DMA((N_DEV - 1,)),
            pltpu.SemaphoreType.DMA((N_DEV - 1,)),
        ],
        compiler_params=pltpu.CompilerParams(
            collective_id=0,
            vmem_limit_bytes=63 * 1024 * 1024,
        ),
    )(x, Wq, K_ext, V_ext, Wo)


# device time: 133364 ns/iter; 1.3736x vs baseline; 1.1255x over previous
import jax
import jax.numpy as jnp
from jax import lax
from jax.experimental import pallas as pl
from jax.experimental.pallas import tpu as pltpu

N_DEV = 4
SQ = 1024
SKV = 1024
HQ_LOCAL = 8
DH = 128
D_MODEL = 1024
BLK = 64
SCALE = 0.08838834764831843
NEG = -1e9


def kernel(x, Wq, K_ext, V_ext, Wo):
    def body(x_ref, wq_ref, k_ref, v_ref, wo_ref, out_ref,
             kv_buf, kv_send_buf, stage, q_buf,
             comm, ar_bf, ag_comm, ag0_bf,
             kv_send_sems, kv_recv_sems, stage_sems,
             relay_recv_sems, relay_fwd_sems,
             ar_send_sems, ar_recv_sems, ag_send_sems, ag_recv_sems):
        my = lax.axis_index("i")
        right = (my + 1) % N_DEV

        bsem = pltpu.get_barrier_semaphore()
        for d in (1, 2, 3):
            pl.semaphore_signal(
                bsem, inc=1,
                device_id=((my + d) % N_DEV,),
                device_id_type=pl.DeviceIdType.MESH,
            )
        pl.semaphore_wait(bsem, N_DEV - 1)

        def relay_view(t):
            return kv_send_buf.at[t, :, 0:HQ_LOCAL, :]

        def kv_rdma(j, t):
            if j == 2:
                dst = relay_view(t)
                dev = 1 if t == 0 else 3
                recv = relay_recv_sems.at[0]
            else:
                dst = kv_buf.at[t]
                dev = j
                recv = kv_recv_sems.at[t]
            return pltpu.make_async_remote_copy(
                src_ref=kv_send_buf.at[t, :, (j - 1) * HQ_LOCAL:j * HQ_LOCAL, :],
                dst_ref=dst,
                send_sem=kv_send_sems.at[j - 1, t],
                recv_sem=recv,
                device_id=(dev,),
                device_id_type=pl.DeviceIdType.MESH,
            )

        def fwd_rdma(t):
            return pltpu.make_async_remote_copy(
                src_ref=relay_view(t),
                dst_ref=kv_buf.at[t],
                send_sem=relay_fwd_sems.at[0],
                recv_sem=kv_recv_sems.at[t],
                device_id=(2,),
                device_id_type=pl.DeviceIdType.MESH,
            )

        @pl.when(my == 0)
        def _():
            jobs = [(2, 0), (2, 1), (1, 0), (3, 0), (1, 1), (3, 1),
                    (0, 0), (0, 1)]

            def stage_dma(idx, slot):
                j, t = jobs[idx]
                src = k_ref if t == 0 else v_ref
                return pltpu.make_async_copy(
                    src.at[0, :, j * HQ_LOCAL:(j + 1) * HQ_LOCAL, :],
                    stage.at[slot],
                    stage_sems.at[slot],
                )

            stage_dma(0, 0).start()
            stage_dma(1, 1).start()
            for idx, (j, t) in enumerate(jobs):
                slot = idx % 2
                stage_dma(idx, slot).wait()
                bf = stage[slot].astype(jnp.bfloat16)
                if j == 0:
                    kv_buf[t] = bf
                else:
                    kv_send_buf[t, :, (j - 1) * HQ_LOCAL:j * HQ_LOCAL, :] = bf
                    kv_rdma(j, t).start()
                if idx + 2 < len(jobs):
                    stage_dma(idx + 2, slot).start()

        @pl.when(my == 1)
        def _():
            kv_rdma(2, 0).wait_recv()
            fwd_rdma(0).start()

        @pl.when(my == 3)
        def _():
            kv_rdma(2, 1).wait_recv()
            fwd_rdma(1).start()

        q_buf[...] = jnp.dot(x_ref[0], wq_ref[...],
                             preferred_element_type=jnp.float32) * SCALE

        @pl.when(my != 0)
        def _():
            for t in (0, 1):
                kv_rdma(1, t).wait_recv()

        @pl.when(my == 0)
        def _():
            for j in (1, 2, 3):
                for t in (0, 1):
                    kv_rdma(j, t).wait_send()

        @pl.when(my == 1)
        def _():
            fwd_rdma(0).wait_send()

        @pl.when(my == 3)
        def _():
            fwd_rdma(1).wait_send()

        rows = SQ // N_DEV
        rows2 = rows // 2
        left = (my - 1) % N_DEV
        DIRS = ((0, 1), (1, -1))

        def ring_dev(d):
            return right if d == 1 else left

        def rs_rdma(di, d, hop):
            return pltpu.make_async_remote_copy(
                src_ref=ar_bf.at[di, hop],
                dst_ref=comm.at[di, hop],
                send_sem=ar_send_sems.at[di, hop],
                recv_sem=ar_recv_sems.at[di, hop],
                device_id=(ring_dev(d),),
                device_id_type=pl.DeviceIdType.MESH,
            )

        def ag_rdma(di, d, hop):
            return pltpu.make_async_remote_copy(
                src_ref=ag0_bf.at[di] if hop == 0 else ag_comm.at[di, hop - 1],
                dst_ref=ag_comm.at[di, hop],
                send_sem=ag_send_sems.at[di, hop],
                recv_sem=ag_recv_sems.at[di, hop],
                device_id=(ring_dev(d),),
                device_id_type=pl.DeviceIdType.MESH,
            )

        def half_ds(g, di):
            return pl.ds(g * rows + di * rows2, rows2)

        qb = lax.broadcasted_iota(jnp.int32, (SQ, SKV), 0) // BLK
        kb = lax.broadcasted_iota(jnp.int32, (SQ, SKV), 1) // BLK
        mask = kb <= qb
        out_ref[0] = jnp.zeros((SQ, D_MODEL), jnp.float32)
        for h in range(HQ_LOCAL):
            qh = q_buf[:, h * DH:(h + 1) * DH]
            kh = kv_buf[0, :, h, :].astype(jnp.float32)
            vh = kv_buf[1, :, h, :].astype(jnp.float32)
            s = lax.dot_general(
                qh, kh, (((1,), (1,)), ((), ())),
                preferred_element_type=jnp.float32,
            )
            e = jnp.exp(jnp.where(mask, s, NEG))
            denom = jnp.sum(e, axis=1, keepdims=True)
            ctx_h = jnp.dot(
                e, vh, preferred_element_type=jnp.float32,
            ) * (1.0 / denom)
            out_ref[0] = out_ref[0] + jnp.dot(
                ctx_h, wo_ref[h * DH:(h + 1) * DH, :],
                preferred_element_type=jnp.float32,
            )

        for hop in range(N_DEV - 1):
            for di, d in DIRS:
                s_g = (my - d * hop) % N_DEV
                ar_bf[di, hop] = out_ref[
                    0, half_ds(s_g, di), :
                ].astype(jnp.bfloat16)
                rs_rdma(di, d, hop).start()
            for di, d in DIRS:
                rs_rdma(di, d, hop).wait_recv()
                r_g = (my - d * (hop + 1)) % N_DEV
                out_ref[0, half_ds(r_g, di), :] = (
                    out_ref[0, half_ds(r_g, di), :]
                    + comm[di, hop].astype(jnp.float32)
                )
        for di, d in DIRS:
            ag0_bf[di] = out_ref[
                0, half_ds((my + d) % N_DEV, di), :
            ].astype(jnp.bfloat16)
        for hop in range(N_DEV - 1):
            for di, d in DIRS:
                ag_rdma(di, d, hop).start()
            for di, d in DIRS:
                ag_rdma(di, d, hop).wait_recv()
                g = (my - d * hop) % N_DEV
                out_ref[0, half_ds(g, di), :] = (
                    ag_comm[di, hop].astype(jnp.float32)
                )
        for hop in range(N_DEV - 1):
            for di, d in DIRS:
                rs_rdma(di, d, hop).wait_send()
                ag_rdma(di, d, hop).wait_send()

    return pl.pallas_call(
        body,
        out_shape=jax.ShapeDtypeStruct((1, SQ, D_MODEL), jnp.float32),
        in_specs=[
            pl.BlockSpec(memory_space=pltpu.VMEM),
            pl.BlockSpec(memory_space=pltpu.VMEM),
            pl.BlockSpec(memory_space=pl.ANY),
            pl.BlockSpec(memory_space=pl.ANY),
            pl.BlockSpec(memory_space=pltpu.VMEM),
        ],
        out_specs=pl.BlockSpec(memory_space=pltpu.VMEM),
        scratch_shapes=[
            pltpu.VMEM((2, SKV, HQ_LOCAL, DH), jnp.bfloat16),
            pltpu.VMEM((2, SKV, 3 * HQ_LOCAL, DH), jnp.bfloat16),
            pltpu.VMEM((2, SKV, HQ_LOCAL, DH), jnp.float32),
            pltpu.VMEM((SQ, D_MODEL), jnp.float32),
            pltpu.VMEM((2, N_DEV - 1, SQ // N_DEV // 2, D_MODEL), jnp.bfloat16),
            pltpu.VMEM((2, N_DEV - 1, SQ // N_DEV // 2, D_MODEL), jnp.bfloat16),
            pltpu.VMEM((2, N_DEV - 1, SQ // N_DEV // 2, D_MODEL), jnp.bfloat16),
            pltpu.VMEM((2, SQ // N_DEV // 2, D_MODEL), jnp.bfloat16),
            pltpu.SemaphoreType.DMA((N_DEV - 1, 2)),
            pltpu.SemaphoreType.DMA((2,)),
            pltpu.SemaphoreType.DMA((2,)),
            pltpu.SemaphoreType.DMA((1,)),
            pltpu.SemaphoreType.DMA((1,)),
            pltpu.SemaphoreType.DMA((2, N_DEV - 1)),
            pltpu.SemaphoreType.DMA((2, N_DEV - 1)),
            pltpu.SemaphoreType.DMA((2, N_DEV - 1)),
            pltpu.SemaphoreType.DMA((2, N_DEV - 1)),
        ],
        compiler_params=pltpu.CompilerParams(
            collective_id=0,
            vmem_limit_bytes=63 * 1024 * 1024,
        ),
    )(x, Wq, K_ext, V_ext, Wo)


# device time: 132401 ns/iter; 1.3836x vs baseline; 1.0073x over previous
import jax
import jax.numpy as jnp
from jax import lax
from jax.experimental import pallas as pl
from jax.experimental.pallas import tpu as pltpu

N_DEV = 4
SQ = 1024
SKV = 1024
HQ_LOCAL = 8
DH = 128
D_MODEL = 1024
BLK = 64
SCALE = 0.08838834764831843
NEG = -1e9


def kernel(x, Wq, K_ext, V_ext, Wo):
    def body(x_ref, wq_ref, k_ref, v_ref, wo_ref, out_ref,
             kv_buf, kv_send_buf, stage, q_buf,
             comm, ar_bf, ag_comm, ag0_bf,
             kv_send_sems, kv_recv_sems, stage_sems,
             relay_recv_sems, relay_fwd_sems,
             ar_send_sems, ar_recv_sems, ag_send_sems, ag_recv_sems):
        my = lax.axis_index("i")
        right = (my + 1) % N_DEV

        bsem = pltpu.get_barrier_semaphore()
        for d in (1, 2, 3):
            pl.semaphore_signal(
                bsem, inc=1,
                device_id=((my + d) % N_DEV,),
                device_id_type=pl.DeviceIdType.MESH,
            )
        pl.semaphore_wait(bsem, N_DEV - 1)

        def relay_view(t):
            return kv_send_buf.at[t, :, 0:HQ_LOCAL, :]

        def kv_rdma(j, t):
            if j == 2:
                dst = relay_view(t)
                dev = 1 if t == 0 else 3
                recv = relay_recv_sems.at[0]
            else:
                dst = kv_buf.at[t]
                dev = j
                recv = kv_recv_sems.at[t]
            return pltpu.make_async_remote_copy(
                src_ref=kv_send_buf.at[t, :, (j - 1) * HQ_LOCAL:j * HQ_LOCAL, :],
                dst_ref=dst,
                send_sem=kv_send_sems.at[j - 1, t],
                recv_sem=recv,
                device_id=(dev,),
                device_id_type=pl.DeviceIdType.MESH,
            )

        def fwd_rdma(t):
            return pltpu.make_async_remote_copy(
                src_ref=relay_view(t),
                dst_ref=kv_buf.at[t],
                send_sem=relay_fwd_sems.at[0],
                recv_sem=kv_recv_sems.at[t],
                device_id=(2,),
                device_id_type=pl.DeviceIdType.MESH,
            )

        @pl.when(my == 0)
        def _():
            jobs = [(2, 0), (2, 1), (1, 0), (3, 0), (1, 1), (3, 1),
                    (0, 0), (0, 1)]

            def stage_dma(idx, slot):
                j, t = jobs[idx]
                src = k_ref if t == 0 else v_ref
                return pltpu.make_async_copy(
                    src.at[0, :, j * HQ_LOCAL:(j + 1) * HQ_LOCAL, :],
                    stage.at[slot],
                    stage_sems.at[slot],
                )

            stage_dma(0, 0).start()
            stage_dma(1, 1).start()
            for idx, (j, t) in enumerate(jobs):
                slot = idx % 2
                stage_dma(idx, slot).wait()
                bf = stage[slot].astype(jnp.bfloat16)
                if j == 0:
                    kv_buf[t] = bf
                else:
                    kv_send_buf[t, :, (j - 1) * HQ_LOCAL:j * HQ_LOCAL, :] = bf
                    kv_rdma(j, t).start()
                if idx + 2 < len(jobs):
                    stage_dma(idx + 2, slot).start()

        @pl.when(my == 1)
        def _():
            kv_rdma(2, 0).wait_recv()
            fwd_rdma(0).start()

        @pl.when(my == 3)
        def _():
            kv_rdma(2, 1).wait_recv()
            fwd_rdma(1).start()

        q_buf[...] = jnp.dot(x_ref[0], wq_ref[...],
                             preferred_element_type=jnp.float32) * SCALE

        @pl.when(my != 0)
        def _():
            for t in (0, 1):
                kv_rdma(1, t).wait_recv()

        @pl.when(my == 0)
        def _():
            for j in (1, 2, 3):
                for t in (0, 1):
                    kv_rdma(j, t).wait_send()

        @pl.when(my == 1)
        def _():
            fwd_rdma(0).wait_send()

        @pl.when(my == 3)
        def _():
            fwd_rdma(1).wait_send()

        rows = SQ // N_DEV
        rows2 = rows // 2
        left = (my - 1) % N_DEV
        DIRS = ((0, 1), (1, -1))

        def ring_dev(d):
            return right if d == 1 else left

        def rs_rdma(di, d, hop):
            return pltpu.make_async_remote_copy(
                src_ref=ar_bf.at[di, hop],
                dst_ref=comm.at[di, hop],
                send_sem=ar_send_sems.at[di, hop],
                recv_sem=ar_recv_sems.at[di, hop],
                device_id=(ring_dev(d),),
                device_id_type=pl.DeviceIdType.MESH,
            )

        def ag_rdma(di, d, hop):
            return pltpu.make_async_remote_copy(
                src_ref=ag0_bf.at[di] if hop == 0 else ag_comm.at[di, hop - 1],
                dst_ref=ag_comm.at[di, hop],
                send_sem=ag_send_sems.at[di, hop],
                recv_sem=ag_recv_sems.at[di, hop],
                device_id=(ring_dev(d),),
                device_id_type=pl.DeviceIdType.MESH,
            )

        def half_ds(g, di):
            return pl.ds(g * rows + di * rows2, rows2)

        HALF = SQ // 2
        masks = []
        for b, kvlen in ((0, HALF), (1, SKV)):
            qb = (
                lax.broadcasted_iota(jnp.int32, (HALF, kvlen), 0) + b * HALF
            ) // BLK
            kb = lax.broadcasted_iota(jnp.int32, (HALF, kvlen), 1) // BLK
            masks.append(kb <= qb)
        out_ref[0] = jnp.zeros((SQ, D_MODEL), jnp.float32)
        for h in range(HQ_LOCAL):
            kh = kv_buf[0, :, h, :].astype(jnp.float32)
            vh = kv_buf[1, :, h, :].astype(jnp.float32)
            for b, kvlen in ((0, HALF), (1, SKV)):
                qh = q_buf[b * HALF:(b + 1) * HALF,
                           h * DH:(h + 1) * DH]
                s = lax.dot_general(
                    qh, kh[:kvlen], (((1,), (1,)), ((), ())),
                    preferred_element_type=jnp.float32,
                )
                e = jnp.exp(jnp.where(masks[b], s, NEG))
                denom = jnp.sum(e, axis=1, keepdims=True)
                ctx_h = jnp.dot(
                    e, vh[:kvlen], preferred_element_type=jnp.float32,
                ) * (1.0 / denom)
                rsl = pl.ds(b * HALF, HALF)
                out_ref[0, rsl, :] = out_ref[0, rsl, :] + jnp.dot(
                    ctx_h, wo_ref[h * DH:(h + 1) * DH, :],
                    preferred_element_type=jnp.float32,
                )

        for hop in range(N_DEV - 1):
            for di, d in DIRS:
                s_g = (my - d * hop) % N_DEV
                ar_bf[di, hop] = out_ref[
                    0, half_ds(s_g, di), :
                ].astype(jnp.bfloat16)
                rs_rdma(di, d, hop).start()
            for di, d in DIRS:
                rs_rdma(di, d, hop).wait_recv()
                r_g = (my - d * (hop + 1)) % N_DEV
                out_ref[0, half_ds(r_g, di), :] = (
                    out_ref[0, half_ds(r_g, di), :]
                    + comm[di, hop].astype(jnp.float32)
                )
        for di, d in DIRS:
            ag0_bf[di] = out_ref[
                0, half_ds((my + d) % N_DEV, di), :
            ].astype(jnp.bfloat16)
        for hop in range(N_DEV - 1):
            for di, d in DIRS:
                ag_rdma(di, d, hop).start()
            for di, d in DIRS:
                ag_rdma(di, d, hop).wait_recv()
                g = (my - d * hop) % N_DEV
                out_ref[0, half_ds(g, di), :] = (
                    ag_comm[di, hop].astype(jnp.float32)
                )
        for hop in range(N_DEV - 1):
            for di, d in DIRS:
                rs_rdma(di, d, hop).wait_send()
                ag_rdma(di, d, hop).wait_send()

    return pl.pallas_call(
        body,
        out_shape=jax.ShapeDtypeStruct((1, SQ, D_MODEL), jnp.float32),
        in_specs=[
            pl.BlockSpec(memory_space=pltpu.VMEM),
            pl.BlockSpec(memory_space=pltpu.VMEM),
            pl.BlockSpec(memory_space=pl.ANY),
            pl.BlockSpec(memory_space=pl.ANY),
            pl.BlockSpec(memory_space=pltpu.VMEM),
        ],
        out_specs=pl.BlockSpec(memory_space=pltpu.VMEM),
        scratch_shapes=[
            pltpu.VMEM((2, SKV, HQ_LOCAL, DH), jnp.bfloat16),
            pltpu.VMEM((2, SKV, 3 * HQ_LOCAL, DH), jnp.bfloat16),
            pltpu.VMEM((2, SKV, HQ_LOCAL, DH), jnp.float32),
            pltpu.VMEM((SQ, D_MODEL), jnp.float32),
            pltpu.VMEM((2, N_DEV - 1, SQ // N_DEV // 2, D_MODEL), jnp.bfloat16),
            pltpu.VMEM((2, N_DEV - 1, SQ // N_DEV // 2, D_MODEL), jnp.bfloat16),
            pltpu.VMEM((2, N_DEV - 1, SQ // N_DEV // 2, D_MODEL), jnp.bfloat16),
            pltpu.VMEM((2, SQ // N_DEV // 2, D_MODEL), jnp.bfloat16),
            pltpu.SemaphoreType.DMA((N_DEV - 1, 2)),
            pltpu.SemaphoreType.DMA((2,)),
            pltpu.SemaphoreType.DMA((2,)),
            pltpu.SemaphoreType.DMA((1,)),
            pltpu.SemaphoreType.DMA((1,)),
            pltpu.SemaphoreType.DMA((2, N_DEV - 1)),
            pltpu.SemaphoreType.DMA((2, N_DEV - 1)),
            pltpu.SemaphoreType.DMA((2, N_DEV - 1)),
            pltpu.SemaphoreType.DMA((2, N_DEV - 1)),
        ],
        compiler_params=pltpu.CompilerParams(
            collective_id=0,
            vmem_limit_bytes=63 * 1024 * 1024,
        ),
    )(x, Wq, K_ext, V_ext, Wo)
